# Initial kernel scaffold; baseline (speedup 1.0000x reference)
import jax
import jax.numpy as jnp
from jax.experimental import pallas as pl


def kernel(input, table):
    raise NotImplementedError("write your pallas kernel here")


# speedup vs baseline: 4.8098x; 4.8098x over previous
import functools

import jax
import jax.numpy as jnp
from jax import lax
from jax.experimental import pallas as pl
from jax.experimental.pallas import tpu as pltpu
from jax.experimental.pallas import tpu_sc as plsc

_VOCAB = 1000000
_EMBED = 32
_BATCH = 16384
_HIST = 200

_NC = 2
_NS = 16
_NW = _NC * _NS

_CHUNK = 128
_B_TOTAL = _BATCH * _HIST
_N_CHUNKS = _B_TOTAL // _CHUNK
_CHUNKS_PER_W = _N_CHUNKS // _NW
_K = 8
_STEPS = _CHUNKS_PER_W // _K


@functools.partial(
    pl.kernel,
    out_type=jax.ShapeDtypeStruct((_N_CHUNKS, _CHUNK, _EMBED), jnp.float32),
    mesh=plsc.VectorSubcoreMesh(core_axis_name="c", subcore_axis_name="s"),
    scratch_types=[
        pltpu.VMEM((_K, _CHUNK), jnp.int32),
        pltpu.VMEM((_K, _CHUNK, _EMBED), jnp.float32),
        pltpu.SemaphoreType.DMA,
    ],
    compiler_params=pltpu.CompilerParams(use_tc_tiling_on_sc=False),
)
def _gather_kernel(idx_hbm, table_hbm, out_hbm, idx_v, rows_v, sem):
    wid = lax.axis_index("s") * _NC + lax.axis_index("c")
    base0 = wid * _CHUNKS_PER_W

    def step(g, carry):
        base = base0 + g * _K
        pltpu.sync_copy(idx_hbm.at[pl.ds(base, _K)], idx_v)
        for j in range(_K):
            pltpu.async_copy(table_hbm.at[idx_v.at[j]], rows_v.at[j], sem)
        for j in range(_K):
            pltpu.make_async_copy(
                table_hbm.at[idx_v.at[j]], rows_v.at[j], sem
            ).wait()
        pltpu.sync_copy(rows_v, out_hbm.at[pl.ds(base, _K)])
        return carry

    lax.fori_loop(0, _STEPS, step, 0)


def kernel(input, table):
    idx = input.reshape(_N_CHUNKS, _CHUNK).astype(jnp.int32)
    out = _gather_kernel(idx, table)
    return out.reshape(_BATCH, _HIST, _EMBED)


# speedup vs baseline: 5.0478x; 1.0495x over previous
import functools

import jax
import jax.numpy as jnp
from jax import lax
from jax.experimental import pallas as pl
from jax.experimental.pallas import tpu as pltpu
from jax.experimental.pallas import tpu_sc as plsc

_VOCAB = 1000000
_EMBED = 32
_BATCH = 16384
_HIST = 200

_NC = 2
_NS = 16
_NW = _NC * _NS

_CHUNK = 128
_B_TOTAL = _BATCH * _HIST
_N_CHUNKS = _B_TOTAL // _CHUNK
_CHUNKS_PER_W = _N_CHUNKS // _NW
_K = 4
_STEPS = _CHUNKS_PER_W // _K
_NB = 4
_ROUNDS = _STEPS // _NB


@functools.partial(
    pl.kernel,
    out_type=jax.ShapeDtypeStruct((_N_CHUNKS, _CHUNK, _EMBED), jnp.float32),
    mesh=plsc.VectorSubcoreMesh(core_axis_name="c", subcore_axis_name="s"),
    scratch_types=[
        pltpu.VMEM((_NB, _K, _CHUNK), jnp.int32),
        pltpu.VMEM((_NB, _K, _CHUNK, _EMBED), jnp.float32),
        [pltpu.SemaphoreType.DMA] * _NB,
        [pltpu.SemaphoreType.DMA] * _NB,
    ],
    compiler_params=pltpu.CompilerParams(use_tc_tiling_on_sc=False),
)
def _gather_kernel(idx_hbm, table_hbm, out_hbm, idx_v, rows_v, gsem, wsem):
    wid = lax.axis_index("s") * _NC + lax.axis_index("c")
    base0 = wid * _CHUNKS_PER_W

    def fire_gathers(s, b):
        base = base0 + s * _K
        pltpu.sync_copy(idx_hbm.at[pl.ds(base, _K)], idx_v.at[b])
        for j in range(_K):
            pltpu.async_copy(table_hbm.at[idx_v.at[b, j]], rows_v.at[b, j], gsem[b])

    def drain_gathers(b):
        for j in range(_K):
            pltpu.make_async_copy(
                table_hbm.at[idx_v.at[b, j]], rows_v.at[b, j], gsem[b]
            ).wait()

    def fire_write(s, b):
        base = base0 + s * _K
        pltpu.async_copy(rows_v.at[b], out_hbm.at[pl.ds(base, _K)], wsem[b])

    def drain_write(b):
        pltpu.make_async_copy(
            rows_v.at[b], out_hbm.at[pl.ds(base0, _K)], wsem[b]
        ).wait()

    def do_step(s, b, prefetch, prefetch_drains):
        if prefetch:
            b2 = (b + 2) % _NB
            if prefetch_drains:
                drain_write(b2)
            fire_gathers(s + 2, b2)
        drain_gathers(b)
        fire_write(s, b)

    fire_gathers(0, 0)
    fire_gathers(1, 1)

    for b in range(_NB):
        do_step(b, b, prefetch=True, prefetch_drains=(b >= 2))

    def round_body(t, carry):
        s0 = t * _NB
        for b in range(_NB):
            do_step(s0 + b, b, prefetch=True, prefetch_drains=True)
        return carry

    lax.fori_loop(1, _ROUNDS - 1, round_body, 0)

    s0 = (_ROUNDS - 1) * _NB
    for b in range(_NB):
        do_step(s0 + b, b, prefetch=(b < 2), prefetch_drains=True)

    for b in range(_NB):
        drain_write(b)


def kernel(input, table):
    idx = input.reshape(_N_CHUNKS, _CHUNK).astype(jnp.int32)
    out = _gather_kernel(idx, table)
    return out.reshape(_BATCH, _HIST, _EMBED)


# speedup vs baseline: 5.0480x; 1.0000x over previous
import functools

import jax
import jax.numpy as jnp
from jax import lax
from jax.experimental import pallas as pl
from jax.experimental.pallas import tpu as pltpu
from jax.experimental.pallas import tpu_sc as plsc

_VOCAB = 1000000
_EMBED = 32
_BATCH = 16384
_HIST = 200

_NC = 2
_NS = 16
_NW = _NC * _NS

_CHUNK = 512
_B_TOTAL = _BATCH * _HIST
_N_CHUNKS = _B_TOTAL // _CHUNK
_CHUNKS_PER_W = _N_CHUNKS // _NW
_K = 1
_STEPS = _CHUNKS_PER_W // _K
_NB = 4
_ROUNDS = _STEPS // _NB


@functools.partial(
    pl.kernel,
    out_type=jax.ShapeDtypeStruct((_N_CHUNKS, _CHUNK, _EMBED), jnp.float32),
    mesh=plsc.VectorSubcoreMesh(core_axis_name="c", subcore_axis_name="s"),
    scratch_types=[
        pltpu.VMEM((_NB, _K, _CHUNK), jnp.int32),
        pltpu.VMEM((_NB, _K, _CHUNK, _EMBED), jnp.float32),
        [pltpu.SemaphoreType.DMA] * _NB,
        [pltpu.SemaphoreType.DMA] * _NB,
    ],
    compiler_params=pltpu.CompilerParams(use_tc_tiling_on_sc=False),
)
def _gather_kernel(idx_hbm, table_hbm, out_hbm, idx_v, rows_v, gsem, wsem):
    wid = lax.axis_index("s") * _NC + lax.axis_index("c")
    base0 = wid * _CHUNKS_PER_W

    def fire_gathers(s, b):
        base = base0 + s * _K
        pltpu.sync_copy(idx_hbm.at[pl.ds(base, _K)], idx_v.at[b])
        for j in range(_K):
            pltpu.async_copy(table_hbm.at[idx_v.at[b, j]], rows_v.at[b, j], gsem[b])

    def drain_gathers(b):
        for j in range(_K):
            pltpu.make_async_copy(
                table_hbm.at[idx_v.at[b, j]], rows_v.at[b, j], gsem[b]
            ).wait()

    def fire_write(s, b):
        base = base0 + s * _K
        pltpu.async_copy(rows_v.at[b], out_hbm.at[pl.ds(base, _K)], wsem[b])

    def drain_write(b):
        pltpu.make_async_copy(
            rows_v.at[b], out_hbm.at[pl.ds(base0, _K)], wsem[b]
        ).wait()

    def do_step(s, b, prefetch, prefetch_drains):
        if prefetch:
            b2 = (b + 2) % _NB
            if prefetch_drains:
                drain_write(b2)
            fire_gathers(s + 2, b2)
        drain_gathers(b)
        fire_write(s, b)

    fire_gathers(0, 0)
    fire_gathers(1, 1)

    for b in range(_NB):
        do_step(b, b, prefetch=True, prefetch_drains=(b >= 2))

    def round_body(t, carry):
        s0 = t * _NB
        for b in range(_NB):
            do_step(s0 + b, b, prefetch=True, prefetch_drains=True)
        return carry

    lax.fori_loop(1, _ROUNDS - 1, round_body, 0)

    s0 = (_ROUNDS - 1) * _NB
    for b in range(_NB):
        do_step(s0 + b, b, prefetch=(b < 2), prefetch_drains=True)

    for b in range(_NB):
        drain_write(b)


def kernel(input, table):
    idx = input.reshape(_N_CHUNKS, _CHUNK).astype(jnp.int32)
    out = _gather_kernel(idx, table)
    return out.reshape(_BATCH, _HIST, _EMBED)


# speedup vs baseline: 5.0501x; 1.0004x over previous
import functools

import jax
import jax.numpy as jnp
from jax import lax
from jax.experimental import pallas as pl
from jax.experimental.pallas import tpu as pltpu
from jax.experimental.pallas import tpu_sc as plsc

_VOCAB = 1000000
_EMBED = 32
_BATCH = 16384
_HIST = 200

_NC = 2
_NS = 16
_NW = _NC * _NS

_ROWS_PER_W = _BATCH // _NW
_K = 4
_STEPS = _ROWS_PER_W // _K
_NB = 4
_ROUNDS = _STEPS // _NB


@functools.partial(
    pl.kernel,
    out_type=jax.ShapeDtypeStruct((_BATCH, _HIST, _EMBED), jnp.float32),
    mesh=plsc.VectorSubcoreMesh(core_axis_name="c", subcore_axis_name="s"),
    scratch_types=[
        pltpu.VMEM((_NB, _K, _HIST), jnp.int32),
        pltpu.VMEM((_NB, _K, _HIST, _EMBED), jnp.float32),
        [pltpu.SemaphoreType.DMA] * _NB,
        [pltpu.SemaphoreType.DMA] * _NB,
    ],
    compiler_params=pltpu.CompilerParams(use_tc_tiling_on_sc=False),
)
def _gather_kernel(idx_hbm, table_hbm, out_hbm, idx_v, rows_v, gsem, wsem):
    wid = lax.axis_index("s") * _NC + lax.axis_index("c")
    base0 = wid * _ROWS_PER_W

    def fire_gathers(s, b):
        base = base0 + s * _K
        pltpu.sync_copy(idx_hbm.at[pl.ds(base, _K)], idx_v.at[b])
        for j in range(_K):
            pltpu.async_copy(table_hbm.at[idx_v.at[b, j]], rows_v.at[b, j], gsem[b])

    def drain_gathers(b):
        for j in range(_K):
            pltpu.make_async_copy(
                table_hbm.at[idx_v.at[b, j]], rows_v.at[b, j], gsem[b]
            ).wait()

    def fire_write(s, b):
        base = base0 + s * _K
        pltpu.async_copy(rows_v.at[b], out_hbm.at[pl.ds(base, _K)], wsem[b])

    def drain_write(b):
        pltpu.make_async_copy(
            rows_v.at[b], out_hbm.at[pl.ds(base0, _K)], wsem[b]
        ).wait()

    def do_step(s, b, prefetch, prefetch_drains):
        if prefetch:
            b2 = (b + 2) % _NB
            if prefetch_drains:
                drain_write(b2)
            fire_gathers(s + 2, b2)
        drain_gathers(b)
        fire_write(s, b)

    fire_gathers(0, 0)
    fire_gathers(1, 1)

    for b in range(_NB):
        do_step(b, b, prefetch=True, prefetch_drains=(b >= 2))

    def round_body(t, carry):
        s0 = t * _NB
        for b in range(_NB):
            do_step(s0 + b, b, prefetch=True, prefetch_drains=True)
        return carry

    lax.fori_loop(1, _ROUNDS - 1, round_body, 0)

    s0 = (_ROUNDS - 1) * _NB
    for b in range(_NB):
        do_step(s0 + b, b, prefetch=(b < 2), prefetch_drains=True)

    for b in range(_NB):
        drain_write(b)


def kernel(input, table):
    return _gather_kernel(input.astype(jnp.int32), table)


# speedup vs baseline: 10.9109x; 2.1605x over previous
import functools

import jax
import jax.numpy as jnp
from jax import lax
from jax.experimental import pallas as pl
from jax.experimental.pallas import tpu as pltpu
from jax.experimental.pallas import tpu_sc as plsc

_VOCAB = 1000000
_EMBED = 32
_BATCH = 16384
_HIST = 200

_NC = 2
_NS = 16
_NW = _NC * _NS

_ROWS_PER_W = _BATCH // _NW
_K = 4
_STEPS = _ROWS_PER_W // _K
_NB = 4
_ROUNDS = _STEPS // _NB
_T = _HIST * _EMBED
_TOKENS = _BATCH * _HIST


@functools.partial(
    pl.kernel,
    out_type=jax.ShapeDtypeStruct((_TOKENS, _EMBED), jnp.float32),
    mesh=plsc.VectorSubcoreMesh(core_axis_name="c", subcore_axis_name="s"),
    scratch_types=[
        pltpu.VMEM((_NB, _K, _HIST), jnp.int32),
        pltpu.VMEM((_NB, _K * _HIST, _EMBED), jnp.float32),
        [pltpu.SemaphoreType.DMA] * _NB,
        [pltpu.SemaphoreType.DMA] * _NB,
    ],
    compiler_params=pltpu.CompilerParams(use_tc_tiling_on_sc=False),
)
def _gather_kernel(idx_hbm, table_hbm, out_hbm, idx_v, rows_v, gsem, wsem):
    wid = lax.axis_index("s") * _NC + lax.axis_index("c")
    base0 = wid * _ROWS_PER_W

    def fire_gathers(s, b):
        base = base0 + s * _K
        pltpu.sync_copy(idx_hbm.at[pl.ds(base, _K)], idx_v.at[b])
        for j in range(_K):
            pltpu.async_copy(
                table_hbm.at[idx_v.at[b, j]],
                rows_v.at[b, pl.ds(j * _HIST, _HIST)],
                gsem[b],
            )

    def drain_gathers(b):
        for j in range(_K):
            pltpu.make_async_copy(
                table_hbm.at[idx_v.at[b, j]],
                rows_v.at[b, pl.ds(j * _HIST, _HIST)],
                gsem[b],
            ).wait()

    def fire_write(s, b):
        base = base0 + s * _K
        pltpu.async_copy(
            rows_v.at[b], out_hbm.at[pl.ds(base * _HIST, _K * _HIST)], wsem[b]
        )

    def drain_write(b):
        pltpu.make_async_copy(
            rows_v.at[b], out_hbm.at[pl.ds(base0 * _HIST, _K * _HIST)], wsem[b]
        ).wait()

    def do_step(s, b, prefetch, prefetch_drains):
        if prefetch:
            b2 = (b + 2) % _NB
            if prefetch_drains:
                drain_write(b2)
            fire_gathers(s + 2, b2)
        drain_gathers(b)
        fire_write(s, b)

    fire_gathers(0, 0)
    fire_gathers(1, 1)

    for b in range(_NB):
        do_step(b, b, prefetch=True, prefetch_drains=(b >= 2))

    def round_body(t, carry):
        s0 = t * _NB
        for b in range(_NB):
            do_step(s0 + b, b, prefetch=True, prefetch_drains=True)
        return carry

    lax.fori_loop(1, _ROUNDS - 1, round_body, 0)

    s0 = (_ROUNDS - 1) * _NB
    for b in range(_NB):
        do_step(s0 + b, b, prefetch=(b < 2), prefetch_drains=True)

    for b in range(_NB):
        drain_write(b)


_BB = 128
_IN_ROWS = _BB * _T // 128


def _transpose_body(x_ref, o_ref):
    o_ref[...] = x_ref[...].reshape(_BB, _T).T


_transpose_tc = pl.pallas_call(
    _transpose_body,
    grid=(_BATCH // _BB,),
    in_specs=[pl.BlockSpec((_IN_ROWS, 128), lambda i: (i, 0))],
    out_specs=pl.BlockSpec((_T, _BB), lambda i: (0, i)),
    out_shape=jax.ShapeDtypeStruct((_T, _BATCH), jnp.float32),
)


def kernel(input, table):
    flat = _gather_kernel(input.astype(jnp.int32), table)
    wide = flat.reshape(_BATCH * _T // 128, 128)
    out_t = _transpose_tc(wide)
    out_t = out_t.reshape(_HIST, _EMBED, _BATCH)
    return out_t.transpose(2, 0, 1)
